# baseline (speedup 1.0000x reference)
import functools

import jax
import jax.numpy as jnp
from jax import lax
from jax.experimental import pallas as pl
from jax.experimental.pallas import tpu as pltpu
from jax.experimental.pallas import tpu_sc as plsc

B, L, V, D, H = 16384, 50, 1000000, 128, 32
NC, NS = 2, 16
NW = NC * NS
GP = 2
IPG = GP * L
NG = B // GP
GPW = NG // NW
LANES = D // 16


UNROLL = 10
NBUF = 8
NIT = GPW // NBUF
ROWS_IT = NBUF * GP


def _pool_sc(xs1, xs2, embed):
    mesh = plsc.VectorSubcoreMesh(core_axis_name="c", subcore_axis_name="s")

    @functools.partial(
        pl.kernel,
        out_type=jax.ShapeDtypeStruct((2 * B, D), jnp.float32),
        mesh=mesh,
        scratch_types=[
            pltpu.VMEM((2 * NBUF, IPG), jnp.int32),
            [pltpu.VMEM((IPG, D), jnp.float32) for _ in range(NBUF)],
            pltpu.VMEM((ROWS_IT, D), jnp.float32),
            pltpu.SemaphoreType.DMA,
            [pltpu.SemaphoreType.DMA for _ in range(NBUF)],
        ],
    )
    def pool(xs1_hbm, xs2_hbm, emb_hbm, out_hbm, idx_v, rows_bufs, out_v,
             isem, sems):
        wid = lax.axis_index("s") * NC + lax.axis_index("c")
        g0 = wid * GPW

        def start(row, j):
            pltpu.async_copy(emb_hbm.at[idx_v.at[row]], rows_bufs[j], sems[j])

        def wait(j):
            pltpu.make_async_copy(
                emb_hbm.at[idx_v.at[0]], rows_bufs[j], sems[j]).wait()

        def fetch_idx(xs_hbm, chunk, half):
            pltpu.async_copy(
                xs_hbm.at[pl.ds(g0 + chunk * NBUF, NBUF)],
                idx_v.at[pl.ds(half * NBUF, NBUF)], isem)

        def wait_idx():
            pltpu.make_async_copy(
                xs1_hbm.at[pl.ds(0, NBUF)], idx_v.at[pl.ds(0, NBUF)],
                isem).wait()

        def accum(buf, out_row):
            for slot in range(GP):
                base = slot * L

                def racc(r, accs):
                    for dr in range(UNROLL):
                        accs = tuple(
                            accs[c] + buf[base + r * UNROLL + dr,
                                          pl.ds(c * 16, 16)]
                            for c in range(LANES)
                        )
                    return accs

                accs = lax.fori_loop(
                    0, L // UNROLL, racc,
                    tuple(jnp.zeros((16,), jnp.float32) for _ in range(LANES)),
                )
                for c in range(LANES):
                    out_v[out_row + slot, pl.ds(c * 16, 16)] = accs[c]

        def run_tower(p, carry):
            @pl.when(p == 0)
            def _prime1():
                pltpu.sync_copy(xs1_hbm.at[pl.ds(g0, NBUF)],
                                idx_v.at[pl.ds(0, NBUF)])
                fetch_idx(xs1_hbm, 1, 1)

            @pl.when(p == 1)
            def _prime2():
                pltpu.sync_copy(xs2_hbm.at[pl.ds(g0, NBUF)],
                                idx_v.at[pl.ds(0, NBUF)])
                fetch_idx(xs2_hbm, 1, 1)

            for j in range(NBUF):
                start(j, j)
            out0 = p * B + g0 * GP

            def step(k4, carry2):
                nxt = lax.rem(k4 + 1, 2) * NBUF

                @pl.when(k4 < NIT - 1)
                def _wait_idx():
                    wait_idx()

                for j in range(NBUF):
                    wait(j)
                    accum(rows_bufs[j], j * GP)

                    @pl.when(k4 < NIT - 1)
                    def _refill():
                        start(nxt + j, j)

                pltpu.sync_copy(
                    out_v, out_hbm.at[pl.ds(out0 + k4 * ROWS_IT, ROWS_IT)])

                @pl.when(k4 < NIT - 2)
                def _fetch_next():
                    cur = lax.rem(k4, 2) * NBUF

                    @pl.when(p == 0)
                    def _f1():
                        pltpu.async_copy(
                            xs1_hbm.at[pl.ds(g0 + (k4 + 2) * NBUF, NBUF)],
                            idx_v.at[pl.ds(cur, NBUF)], isem)

                    @pl.when(p == 1)
                    def _f2():
                        pltpu.async_copy(
                            xs2_hbm.at[pl.ds(g0 + (k4 + 2) * NBUF, NBUF)],
                            idx_v.at[pl.ds(cur, NBUF)], isem)

                return carry2

            lax.fori_loop(0, NIT, step, 0)
            return carry

        lax.fori_loop(0, 2, run_tower, 0)

    return pool(xs1, xs2, embed)


BT = 2048


def _mlp_tc(pooled, w1t, b1v, b1h, w2t, b2v, b2h):
    def body(p1_ref, p2_ref, w1_ref, b1v_ref, b1h_ref, w2_ref, b2v_ref,
             b2h_ref, o_ref):
        h1 = jnp.tanh(p1_ref[...] + b1v_ref[...])
        a1 = jnp.tanh(
            jnp.dot(h1, w1_ref[...], preferred_element_type=jnp.float32)
            + b1h_ref[...])
        h2 = jnp.tanh(p2_ref[...] + b2v_ref[...])
        a2 = jnp.tanh(
            jnp.dot(h2, w2_ref[...], preferred_element_type=jnp.float32)
            + b2h_ref[...])
        s = jnp.sum(a1 * a2, axis=1)
        o_ref[...] = (1.0 / (1.0 + jnp.exp(-s)))[None, :]

    return pl.pallas_call(
        body,
        grid=(B // BT,),
        in_specs=[
            pl.BlockSpec((BT, D), lambda i: (i, 0)),
            pl.BlockSpec((BT, D), lambda i: (i + B // BT, 0)),
            pl.BlockSpec((D, H), lambda i: (0, 0)),
            pl.BlockSpec((1, D), lambda i: (0, 0)),
            pl.BlockSpec((1, H), lambda i: (0, 0)),
            pl.BlockSpec((D, H), lambda i: (0, 0)),
            pl.BlockSpec((1, D), lambda i: (0, 0)),
            pl.BlockSpec((1, H), lambda i: (0, 0)),
        ],
        out_specs=pl.BlockSpec((1, BT), lambda i: (0, i)),
        out_shape=jax.ShapeDtypeStruct((1, B), jnp.float32),
    )(pooled, pooled, w1t, b1v, b1h, w2t, b2v, b2h)


def kernel(x1, x2, embed, t1_bias1, t1_W, t1_b, t2_bias1, t2_W, t2_b):
    xs1 = x1.astype(jnp.int32).reshape(NG, IPG)
    xs2 = x2.astype(jnp.int32).reshape(NG, IPG)
    pooled = _pool_sc(xs1, xs2, embed)
    out = _mlp_tc(
        pooled,
        t1_W.T, t1_bias1[None, :], t1_b[None, :],
        t2_W.T, t2_bias1[None, :], t2_b[None, :],
    )
    return out.reshape(B)


# speedup vs baseline: 1.1423x; 1.1423x over previous
import functools

import jax
import jax.numpy as jnp
from jax import lax
from jax.experimental import pallas as pl
from jax.experimental.pallas import tpu as pltpu
from jax.experimental.pallas import tpu_sc as plsc

B, L, V, D, H = 16384, 50, 1000000, 128, 32
NC, NS = 2, 16
NW = NC * NS
GP = 2
IPG = GP * L
NG = B // GP
GPW = NG // NW
LANES = D // 16


UNROLL = 10
NBUF = 8
NIT = GPW // NBUF
ROWS_IT = NBUF * GP


def _pool_sc(xs1, xs2, embed):
    mesh = plsc.VectorSubcoreMesh(core_axis_name="c", subcore_axis_name="s")

    @functools.partial(
        pl.kernel,
        out_type=jax.ShapeDtypeStruct((2 * B, D), jnp.float32),
        mesh=mesh,
        scratch_types=[
            pltpu.VMEM((3 * NBUF, IPG), jnp.int32),
            [pltpu.VMEM((IPG, D), jnp.float32) for _ in range(NBUF)],
            pltpu.VMEM((ROWS_IT, D), jnp.float32),
            pltpu.SemaphoreType.DMA,
            [pltpu.SemaphoreType.DMA for _ in range(NBUF)],
        ],
    )
    def pool(xs1_hbm, xs2_hbm, emb_hbm, out_hbm, idx_v, rows_bufs, out_v,
             isem, sems):
        wid = lax.axis_index("s") * NC + lax.axis_index("c")
        g0 = wid * GPW

        def start(row, j):
            pltpu.async_copy(emb_hbm.at[idx_v.at[row]], rows_bufs[j], sems[j])

        def wait(j):
            pltpu.make_async_copy(
                emb_hbm.at[idx_v.at[0]], rows_bufs[j], sems[j]).wait()

        def fetch_idx(xs_hbm, chunk, half):
            pltpu.async_copy(
                xs_hbm.at[pl.ds(g0 + chunk * NBUF, NBUF)],
                idx_v.at[pl.ds(half * NBUF, NBUF)], isem)

        def wait_idx():
            pltpu.make_async_copy(
                xs1_hbm.at[pl.ds(0, NBUF)], idx_v.at[pl.ds(0, NBUF)],
                isem).wait()

        def accum(buf, out_row):
            for slot in range(GP):
                base = slot * L

                def racc(r, accs):
                    for dr in range(UNROLL):
                        accs = tuple(
                            accs[c] + buf[base + r * UNROLL + dr,
                                          pl.ds(c * 16, 16)]
                            for c in range(LANES)
                        )
                    return accs

                accs = lax.fori_loop(
                    0, L // UNROLL, racc,
                    tuple(jnp.zeros((16,), jnp.float32) for _ in range(LANES)),
                )
                for c in range(LANES):
                    out_v[out_row + slot, pl.ds(c * 16, 16)] = accs[c]

        def run_tower(p, carry):
            @pl.when(p == 0)
            def _prime1():
                pltpu.sync_copy(xs1_hbm.at[pl.ds(g0, NBUF)],
                                idx_v.at[pl.ds(0, NBUF)])
                fetch_idx(xs1_hbm, 1, 1)
                fetch_idx(xs1_hbm, 2, 2)

            @pl.when(p == 1)
            def _prime2():
                pltpu.sync_copy(xs2_hbm.at[pl.ds(g0, NBUF)],
                                idx_v.at[pl.ds(0, NBUF)])
                fetch_idx(xs2_hbm, 1, 1)
                fetch_idx(xs2_hbm, 2, 2)

            for j in range(NBUF):
                start(j, j)
            out0 = p * B + g0 * GP

            def step(k4, carry2):
                nxt = lax.rem(k4 + 1, 3) * NBUF

                @pl.when(k4 < NIT - 1)
                def _wait_idx():
                    wait_idx()

                for j in range(NBUF):
                    wait(j)
                    accum(rows_bufs[j], j * GP)

                    @pl.when(k4 < NIT - 1)
                    def _refill():
                        start(nxt + j, j)

                pltpu.sync_copy(
                    out_v, out_hbm.at[pl.ds(out0 + k4 * ROWS_IT, ROWS_IT)])

                @pl.when(k4 < NIT - 3)
                def _fetch_next():
                    slot = lax.rem(k4, 3) * NBUF

                    @pl.when(p == 0)
                    def _f1():
                        pltpu.async_copy(
                            xs1_hbm.at[pl.ds(g0 + (k4 + 3) * NBUF, NBUF)],
                            idx_v.at[pl.ds(slot, NBUF)], isem)

                    @pl.when(p == 1)
                    def _f2():
                        pltpu.async_copy(
                            xs2_hbm.at[pl.ds(g0 + (k4 + 3) * NBUF, NBUF)],
                            idx_v.at[pl.ds(slot, NBUF)], isem)

                return carry2

            lax.fori_loop(0, NIT, step, 0)
            return carry

        lax.fori_loop(0, 2, run_tower, 0)

    return pool(xs1, xs2, embed)


BT = 2048


def _mlp_tc(pooled, w1t, b1v, b1h, w2t, b2v, b2h):
    def body(p1_ref, p2_ref, w1_ref, b1v_ref, b1h_ref, w2_ref, b2v_ref,
             b2h_ref, o_ref):
        h1 = jnp.tanh(p1_ref[...] + b1v_ref[...])
        a1 = jnp.tanh(
            jnp.dot(h1, w1_ref[...], preferred_element_type=jnp.float32)
            + b1h_ref[...])
        h2 = jnp.tanh(p2_ref[...] + b2v_ref[...])
        a2 = jnp.tanh(
            jnp.dot(h2, w2_ref[...], preferred_element_type=jnp.float32)
            + b2h_ref[...])
        s = jnp.sum(a1 * a2, axis=1)
        o_ref[...] = (1.0 / (1.0 + jnp.exp(-s)))[None, :]

    return pl.pallas_call(
        body,
        grid=(B // BT,),
        in_specs=[
            pl.BlockSpec((BT, D), lambda i: (i, 0)),
            pl.BlockSpec((BT, D), lambda i: (i + B // BT, 0)),
            pl.BlockSpec((D, H), lambda i: (0, 0)),
            pl.BlockSpec((1, D), lambda i: (0, 0)),
            pl.BlockSpec((1, H), lambda i: (0, 0)),
            pl.BlockSpec((D, H), lambda i: (0, 0)),
            pl.BlockSpec((1, D), lambda i: (0, 0)),
            pl.BlockSpec((1, H), lambda i: (0, 0)),
        ],
        out_specs=pl.BlockSpec((1, BT), lambda i: (0, i)),
        out_shape=jax.ShapeDtypeStruct((1, B), jnp.float32),
    )(pooled, pooled, w1t, b1v, b1h, w2t, b2v, b2h)


def kernel(x1, x2, embed, t1_bias1, t1_W, t1_b, t2_bias1, t2_W, t2_b):
    xs1 = x1.astype(jnp.int32).reshape(NG, IPG)
    xs2 = x2.astype(jnp.int32).reshape(NG, IPG)
    pooled = _pool_sc(xs1, xs2, embed)
    out = _mlp_tc(
        pooled,
        t1_W.T, t1_bias1[None, :], t1_b[None, :],
        t2_W.T, t2_bias1[None, :], t2_b[None, :],
    )
    return out.reshape(B)


# speedup vs baseline: 1.4547x; 1.2735x over previous
import functools

import jax
import jax.numpy as jnp
from jax import lax
from jax.experimental import pallas as pl
from jax.experimental.pallas import tpu as pltpu
from jax.experimental.pallas import tpu_sc as plsc

B, L, V, D, H = 16384, 50, 1000000, 128, 32
NC, NS = 2, 16
NW = NC * NS
IPG = L
GPW = B // NW
BLK = 32
LANES = D // 16


UNROLL = 10
NBUF = 8
NIT = GPW // NBUF
ITPB = BLK // NBUF


def _pool_sc(xs1, xs2, embed):
    mesh = plsc.VectorSubcoreMesh(core_axis_name="c", subcore_axis_name="s")

    @functools.partial(
        pl.kernel,
        out_type=jax.ShapeDtypeStruct((2 * B, D), jnp.float32),
        mesh=mesh,
        scratch_types=[
            pltpu.VMEM((GPW, IPG), jnp.int32),
            [pltpu.VMEM((IPG, D), jnp.float32) for _ in range(NBUF)],
            pltpu.VMEM((2 * BLK, D), jnp.float32),
            pltpu.SemaphoreType.DMA,
            [pltpu.SemaphoreType.DMA for _ in range(NBUF)],
        ],
    )
    def pool(xs1_hbm, xs2_hbm, emb_hbm, out_hbm, idx_v, rows_bufs, out_v,
             osem, sems):
        wid = lax.axis_index("s") * NC + lax.axis_index("c")
        g0 = wid * GPW

        def start(k, j):
            pltpu.async_copy(emb_hbm.at[idx_v.at[k]], rows_bufs[j], sems[j])

        def wait(j):
            pltpu.make_async_copy(
                emb_hbm.at[idx_v.at[0]], rows_bufs[j], sems[j]).wait()

        def wait_flush():
            pltpu.make_async_copy(
                out_v.at[pl.ds(0, BLK)], out_hbm.at[pl.ds(0, BLK)],
                osem).wait()

        def accum(buf, out_row):

            def racc(r, accs):
                for dr in range(UNROLL):
                    accs = tuple(
                        accs[c] + buf[r * UNROLL + dr, pl.ds(c * 16, 16)]
                        for c in range(LANES)
                    )
                return accs

            accs = lax.fori_loop(
                0, L // UNROLL, racc,
                tuple(jnp.zeros((16,), jnp.float32) for _ in range(LANES)),
            )
            for c in range(LANES):
                out_v[out_row, pl.ds(c * 16, 16)] = accs[c]

        def run_tower(xs_hbm, tower):
            out0 = tower * B + g0
            pltpu.sync_copy(xs_hbm.at[pl.ds(g0, GPW)], idx_v)
            for j in range(NBUF):
                start(j, j)

            def step(k4, carry):
                k = k4 * NBUF
                m = lax.rem(k4, ITPB)
                blk = lax.div(k4, ITPB)
                half = lax.rem(blk, 2) * BLK
                for j in range(NBUF):
                    wait(j)
                    accum(rows_bufs[j], half + m * NBUF + j)

                    @pl.when(k + j + NBUF < GPW)
                    def _refill():
                        start(k + j + NBUF, j)

                @pl.when(m == ITPB - 1)
                def _flush():
                    pltpu.async_copy(
                        out_v.at[pl.ds(half, BLK)],
                        out_hbm.at[pl.ds(out0 + blk * BLK, BLK)], osem)

                    @pl.when(blk >= 1)
                    def _wait_prev():
                        wait_flush()

                return carry

            lax.fori_loop(0, NIT, step, 0)
            wait_flush()

        run_tower(xs1_hbm, 0)
        run_tower(xs2_hbm, 1)

    return pool(xs1, xs2, embed)


BT = 2048


def _mlp_tc(pooled, w1t, b1v, b1h, w2t, b2v, b2h):
    def body(p1_ref, p2_ref, w1_ref, b1v_ref, b1h_ref, w2_ref, b2v_ref,
             b2h_ref, o_ref):
        h1 = jnp.tanh(p1_ref[...] + b1v_ref[...])
        a1 = jnp.tanh(
            jnp.dot(h1, w1_ref[...], preferred_element_type=jnp.float32)
            + b1h_ref[...])
        h2 = jnp.tanh(p2_ref[...] + b2v_ref[...])
        a2 = jnp.tanh(
            jnp.dot(h2, w2_ref[...], preferred_element_type=jnp.float32)
            + b2h_ref[...])
        s = jnp.sum(a1 * a2, axis=1)
        o_ref[...] = (1.0 / (1.0 + jnp.exp(-s)))[None, :]

    return pl.pallas_call(
        body,
        grid=(B // BT,),
        in_specs=[
            pl.BlockSpec((BT, D), lambda i: (i, 0)),
            pl.BlockSpec((BT, D), lambda i: (i + B // BT, 0)),
            pl.BlockSpec((D, H), lambda i: (0, 0)),
            pl.BlockSpec((1, D), lambda i: (0, 0)),
            pl.BlockSpec((1, H), lambda i: (0, 0)),
            pl.BlockSpec((D, H), lambda i: (0, 0)),
            pl.BlockSpec((1, D), lambda i: (0, 0)),
            pl.BlockSpec((1, H), lambda i: (0, 0)),
        ],
        out_specs=pl.BlockSpec((1, BT), lambda i: (0, i)),
        out_shape=jax.ShapeDtypeStruct((1, B), jnp.float32),
    )(pooled, pooled, w1t, b1v, b1h, w2t, b2v, b2h)


def kernel(x1, x2, embed, t1_bias1, t1_W, t1_b, t2_bias1, t2_W, t2_b):
    pooled = _pool_sc(x1.astype(jnp.int32), x2.astype(jnp.int32), embed)
    out = _mlp_tc(
        pooled,
        t1_W.T, t1_bias1[None, :], t1_b[None, :],
        t2_W.T, t2_bias1[None, :], t2_b[None, :],
    )
    return out.reshape(B)
